# baseline (speedup 1.0000x reference)
import functools

import jax
import jax.numpy as jnp
from jax import lax
from jax.experimental import pallas as pl
from jax.experimental.pallas import tpu as pltpu
from jax.experimental.pallas import tpu_sc as plsc

PAD_ID = 0
IGNORE_ID = -100
LANES = 16


def _sc_embed(dec_flat, wte, wpe):
    N = dec_flat.shape[0]
    D = wte.shape[1]
    T = wpe.shape[0]
    info = plsc.get_sparse_core_info()
    nw = info.num_cores * info.num_subcores
    per_w = N // nw
    C = 64
    n_chunks = per_w // C
    mesh = plsc.VectorSubcoreMesh(core_axis_name="c", subcore_axis_name="s")

    @functools.partial(
        pl.kernel,
        mesh=mesh,
        out_type=(
            jax.ShapeDtypeStruct((N, D), jnp.float32),
            jax.ShapeDtypeStruct((N,), jnp.int32),
            jax.ShapeDtypeStruct((N,), jnp.int32),
        ),
        scratch_types=[
            pltpu.VMEM((C,), jnp.int32),
            pltpu.VMEM((C,), jnp.int32),
            pltpu.VMEM((C,), jnp.int32),
            pltpu.VMEM((C, D), jnp.float32),
            pltpu.VMEM((C, D), jnp.float32),
            pltpu.SemaphoreType.DMA,
        ],
    )
    def k(dec_hbm, wte_hbm, wpe_hbm, tok_hbm, din_hbm, keep_hbm,
          dec_v, din_v, keep_v, rows_v, wpe_v, sem):
        wid = lax.axis_index("s") * info.num_cores + lax.axis_index("c")
        base = wid * per_w
        t0 = lax.rem(base, T)

        def chunk_body(c, carry):
            rb = pl.multiple_of(base + c * C, C)
            pltpu.sync_copy(dec_hbm.at[pl.ds(rb, C)], dec_v)
            ign = jnp.full((LANES,), IGNORE_ID, jnp.int32)
            pad = jnp.full((LANES,), PAD_ID, jnp.int32)
            one = jnp.full((LANES,), 1, jnp.int32)
            for i in range(C // LANES):
                sl = pl.ds(i * LANES, LANES)
                v = dec_v[sl]
                m = v != ign
                din_v[sl] = jnp.where(m, v, pad)
                keep_v[sl] = jnp.where(m, one, pad)
            pltpu.sync_copy(din_v, din_hbm.at[pl.ds(rb, C)])
            pltpu.sync_copy(keep_v, keep_hbm.at[pl.ds(rb, C)])
            tb = pl.multiple_of(t0 + c * C, C)
            gather = pltpu.async_copy(wte_hbm.at[din_v], rows_v, sem)
            pltpu.sync_copy(wpe_hbm.at[pl.ds(tb, C)], wpe_v)
            gather.wait()

            def add_row(r, cc):
                for j in range(D // LANES):
                    sl = pl.ds(j * LANES, LANES)
                    rows_v[r, sl] = rows_v[r, sl] + wpe_v[r, sl]
                return cc

            lax.fori_loop(0, C, add_row, 0)
            pltpu.sync_copy(rows_v, tok_hbm.at[pl.ds(rb, C)])
            return carry

        lax.fori_loop(0, n_chunks, chunk_body, 0)

    return k(dec_flat, wte, wpe)


def kernel(enc_hid, dec_or_lab, metadata, wte, wpe):
    B, T = dec_or_lab.shape
    D = wte.shape[1]
    dec_flat = dec_or_lab.reshape(B * T)
    tok, din, keep = _sc_embed(dec_flat, wte, wpe[:T])
    token_emb = tok.reshape(B, T, D)
    keep_b = keep.reshape(B, T).astype(bool)
    dec_in = din.reshape(B, T)
    enc_mask_2d = jnp.ones((B, T), dtype=bool)
    return (enc_hid, token_emb, enc_mask_2d, keep_b, metadata, dec_in, keep_b)
